# Initial kernel scaffold; baseline (speedup 1.0000x reference)
import jax
import jax.numpy as jnp
from jax.experimental import pallas as pl


def kernel(update, index1, index2, params):
    raise NotImplementedError("write your pallas kernel here")


# speedup vs baseline: 2.8335x; 2.8335x over previous
import jax
import jax.numpy as jnp
from jax.experimental import pallas as pl
from jax.experimental.pallas import tpu as pltpu


def _scatter_body(upd_ref, i1_ref, i2_ref, p_ref, o_ref):
    p = p_ref[...]
    row = jax.lax.broadcasted_iota(jnp.int32, p.shape, 0)
    col = jax.lax.broadcasted_iota(jnp.int32, p.shape, 1)
    dep = jax.lax.broadcasted_iota(jnp.int32, p.shape, 2)
    out = p
    for i in range(2):
        m = (row == i1_ref[i]) & (col == (i + 1)) & (dep == i2_ref[i])
        out = jnp.where(m, upd_ref[i], out)
    o_ref[...] = out


def kernel(update, index1, index2, params):
    i1 = index1.astype(jnp.int32)
    i2 = index2.astype(jnp.int32)
    return pl.pallas_call(
        _scatter_body,
        out_shape=jax.ShapeDtypeStruct(params.shape, params.dtype),
        in_specs=[
            pl.BlockSpec(memory_space=pltpu.SMEM),
            pl.BlockSpec(memory_space=pltpu.SMEM),
            pl.BlockSpec(memory_space=pltpu.SMEM),
            pl.BlockSpec(memory_space=pltpu.VMEM),
        ],
        out_specs=pl.BlockSpec(memory_space=pltpu.VMEM),
    )(update, i1, i2, params)
